# baseline (speedup 1.0000x reference)
import functools

import jax
import jax.numpy as jnp
from jax import lax
from jax.experimental import pallas as pl
from jax.experimental.pallas import tpu as pltpu


def _dg(a, b):
    return lax.dot_general(a, b, (((1,), (1,)), ((), ())),
                           preferred_element_type=jnp.float32)


def _ln_rows(z, g, b, eps=1e-5):
    m = jnp.mean(z, axis=-1, keepdims=True)
    v = jnp.mean((z - m) ** 2, axis=-1, keepdims=True)
    return (z - m) * lax.rsqrt(v + eps) * g + b


def _pass_a_body(x_ref, wup_ref, wgate_ref, wpre_ref, wpost_ref, lng_ref,
                 lnb_ref, wrg_ref, wre_ref,
                 hid_ref, pre_ref, ain_ref, aout_ref, ew_ref, loss_ref,
                 load_acc, sq_acc, *, n_tok, ng, gs, n_exp):
    i = pl.program_id(0)
    nprog = pl.num_programs(0)
    x = x_ref[...]

    up = _dg(x, wup_ref[...])
    gate = _dg(x, wgate_ref[...])
    hidden = jax.nn.silu(gate) * up
    hid_ref[...] = hidden

    pre = _dg(x, wpre_ref[...])
    pre_ref[...] = pre
    g = lng_ref[...]
    b = lnb_ref[...]
    ain_ref[...] = _ln_rows(pre, g, b)
    post = _dg(hidden, wpost_ref[...])
    aout_ref[...] = _ln_rows(post, g, b)

    gl = _dg(x, wrg_ref[...])
    ll = _dg(x, wre_ref[...])
    gp = jax.nn.softmax(gl, axis=-1)
    lp = jax.nn.softmax(ll, axis=-1)

    iog = lax.broadcasted_iota(jnp.int32, gp.shape, 1)
    cw = jnp.max(gp, axis=-1, keepdims=True)
    cg = jnp.min(jnp.where(gp == cw, iog, ng), axis=-1, keepdims=True)

    iol = lax.broadcasted_iota(jnp.int32, lp.shape, 1)
    v1 = jnp.max(lp, axis=-1, keepdims=True)
    i1 = jnp.min(jnp.where(lp == v1, iol, gs), axis=-1, keepdims=True)
    lp2 = jnp.where(iol == i1, -1.0, lp)
    v2 = jnp.max(lp2, axis=-1, keepdims=True)
    i2 = jnp.min(jnp.where(lp2 == v2, iol, gs), axis=-1, keepdims=True)

    lsum = v1 + v2 + 1e-7
    f1 = cw * v1 / lsum
    f2 = cw * v2 / lsum

    cols = lax.broadcasted_iota(jnp.int32, (x.shape[0], n_exp), 1)
    g_of = cols // gs
    j_of = cols % gs
    ew = jnp.where(
        g_of == cg,
        jnp.where(j_of == i1, f1, jnp.where(j_of == i2, f2, 0.0)),
        0.0)
    ew_ref[...] = ew

    @pl.when(i == 0)
    def _():
        load_acc[...] = jnp.zeros_like(load_acc)
        sq_acc[...] = jnp.zeros_like(sq_acc)

    load_acc[...] += jnp.sum(ew, axis=0, keepdims=True)
    zpart = (jnp.sum(gl * gl) / (n_tok * ng)
             + jnp.sum(ll * ll) / (n_tok * gs))
    sq_acc[...] += zpart.reshape(1, 1)

    @pl.when(i == nprog - 1)
    def _():
        load = load_acc[...]
        target = jnp.sum(load) / n_exp
        lb = jnp.sum((load - target) ** 2) / n_exp
        loss_ref[...] = 0.001 * (lb + sq_acc[...])


def _pass_b_body(hid_ref, pre_ref, aint_ref, ainf_ref, aoutf_ref, ew_ref,
                 wap_ref, wadp_ref, lneg_ref, lneb_ref, wep_ref, wop_ref,
                 wdown_ref, out_ref, *, n_exp, a_dim):
    h = hid_ref[0]
    ain_i = aint_ref[0]
    ain_b = ainf_ref[0]
    aout_b = aoutf_ref[0]

    aw = _dg(ain_i, aout_b)
    aw = jax.nn.silu(jnp.clip(aw, -5.0, 5.0))
    ad = lax.dot_general(aw, ain_b, (((1,), (0,)), ((), ())),
                         preferred_element_type=jnp.float32)
    h2 = h + 0.1 * _dg(ad, wap_ref[...])
    shared = _dg(h2, wdown_ref[...])

    pre = pre_ref[0]
    zall = _dg(pre, wadp_ref[...])
    ew = ew_ref[0]
    weighted = jnp.zeros((pre.shape[0], a_dim), jnp.float32)
    for e in range(n_exp):
        z = zall[:, e * a_dim:(e + 1) * a_dim]
        zn = _ln_rows(z, lneg_ref[e:e + 1, :], lneb_ref[e:e + 1, :])
        weighted += zn * ew[:, e:e + 1]
    contrib = _dg(_dg(weighted, wep_ref[...]), wop_ref[...])
    wsum = jnp.sum(ew, axis=-1, keepdims=True)
    out_ref[0] = shared * wsum + 0.1 * contrib


def kernel(x, W_up, W_gate, W_down, W_pre, W_post, ln_g, ln_b, W_ap, W_adp,
           lne_g, lne_b, W_ep, W_op, W_rg, W_re):
    B, S, D = x.shape
    H = W_up.shape[0]
    A = W_pre.shape[0]
    E = W_adp.shape[0]
    NG = W_rg.shape[0]
    GS = W_re.shape[0]
    N = B * S
    TA = 512
    TB = 512

    xf = x.reshape(N, D)
    lng2 = ln_g.reshape(1, A)
    lnb2 = ln_b.reshape(1, A)
    wadp_rs = W_adp.reshape(E * A, A)

    const = lambda *_: (0, 0)
    hid, pre, ain, aout, ew, loss = pl.pallas_call(
        functools.partial(_pass_a_body, n_tok=N, ng=NG, gs=GS, n_exp=E),
        grid=(N // TA,),
        in_specs=[
            pl.BlockSpec((TA, D), lambda i: (i, 0)),
            pl.BlockSpec((H, D), const),
            pl.BlockSpec((H, D), const),
            pl.BlockSpec((A, D), const),
            pl.BlockSpec((A, H), const),
            pl.BlockSpec((1, A), const),
            pl.BlockSpec((1, A), const),
            pl.BlockSpec((NG, D), const),
            pl.BlockSpec((GS, D), const),
        ],
        out_specs=[
            pl.BlockSpec((TA, H), lambda i: (i, 0)),
            pl.BlockSpec((TA, A), lambda i: (i, 0)),
            pl.BlockSpec((TA, A), lambda i: (i, 0)),
            pl.BlockSpec((TA, A), lambda i: (i, 0)),
            pl.BlockSpec((TA, E), lambda i: (i, 0)),
            pl.BlockSpec((1, 1), const),
        ],
        out_shape=[
            jax.ShapeDtypeStruct((N, H), jnp.float32),
            jax.ShapeDtypeStruct((N, A), jnp.float32),
            jax.ShapeDtypeStruct((N, A), jnp.float32),
            jax.ShapeDtypeStruct((N, A), jnp.float32),
            jax.ShapeDtypeStruct((N, E), jnp.float32),
            jax.ShapeDtypeStruct((1, 1), jnp.float32),
        ],
        scratch_shapes=[
            pltpu.VMEM((1, E), jnp.float32),
            pltpu.VMEM((1, 1), jnp.float32),
        ],
    )(xf, W_up, W_gate, W_pre, W_post, lng2, lnb2, W_rg, W_re)

    hid3 = hid.reshape(B, S, H)
    pre3 = pre.reshape(B, S, A)
    ain3 = ain.reshape(B, S, A)
    aout3 = aout.reshape(B, S, A)
    ew3 = ew.reshape(B, S, E)

    const3 = lambda b, i: (0, 0)
    out3 = pl.pallas_call(
        functools.partial(_pass_b_body, n_exp=E, a_dim=A),
        grid=(B, S // TB),
        in_specs=[
            pl.BlockSpec((1, TB, H), lambda b, i: (b, i, 0)),
            pl.BlockSpec((1, TB, A), lambda b, i: (b, i, 0)),
            pl.BlockSpec((1, TB, A), lambda b, i: (b, i, 0)),
            pl.BlockSpec((1, S, A), lambda b, i: (b, 0, 0)),
            pl.BlockSpec((1, S, A), lambda b, i: (b, 0, 0)),
            pl.BlockSpec((1, TB, E), lambda b, i: (b, i, 0)),
            pl.BlockSpec((H, A), const3),
            pl.BlockSpec((E * A, A), const3),
            pl.BlockSpec((E, A), const3),
            pl.BlockSpec((E, A), const3),
            pl.BlockSpec((H, A), const3),
            pl.BlockSpec((D, H), const3),
            pl.BlockSpec((D, H), const3),
        ],
        out_specs=pl.BlockSpec((1, TB, D), lambda b, i: (b, i, 0)),
        out_shape=jax.ShapeDtypeStruct((B, S, D), jnp.float32),
    )(hid3, pre3, ain3, ain3, aout3, ew3, W_ap, wadp_rs, lne_g, lne_b,
      W_ep, W_op, W_down)

    return out3, loss[0, 0]


# speedup vs baseline: 1.9419x; 1.9419x over previous
import functools

import jax
import jax.numpy as jnp
from jax import lax
from jax.experimental import pallas as pl
from jax.experimental.pallas import tpu as pltpu


def _dg(a, b):
    return lax.dot_general(a, b, (((1,), (1,)), ((), ())),
                           preferred_element_type=jnp.float32)


def _ln_mm(z, g, b, ones_row, eps=1e-5):
    m = _dg(z, ones_row)
    e2 = _dg(z * z, ones_row)
    v = e2 - m * m
    return (z - m) * lax.rsqrt(v + eps) * g + b


def _fold_body(wdown_ref, wap_ref, wop_ref, wep_ref, wda_ref, woe_ref):
    wda_ref[...] = lax.dot_general(
        wdown_ref[...], wap_ref[...], (((1,), (0,)), ((), ())),
        preferred_element_type=jnp.float32)
    woe_ref[...] = lax.dot_general(
        wop_ref[...], wep_ref[...], (((1,), (0,)), ((), ())),
        preferred_element_type=jnp.float32)


def _pass_a_body(x_ref, wup_ref, wgate_ref, wpre_ref, wpost_ref, lng_ref,
                 lnb_ref, wrg_ref, wre_ref,
                 hid_ref, pre_ref, ain_ref, aout_ref, ew_ref, loss_ref,
                 load_acc, sq_acc, *, n_tok, ng, gs, n_exp):
    i = pl.program_id(0)
    nprog = pl.num_programs(0)
    x = x_ref[...]
    a_dim = wpre_ref.shape[0]
    o_a = jnp.full((1, a_dim), 1.0 / a_dim, jnp.float32)

    up = _dg(x, wup_ref[...])
    gate = _dg(x, wgate_ref[...])
    hidden = jax.nn.silu(gate) * up
    hid_ref[...] = hidden

    pre = _dg(x, wpre_ref[...])
    pre_ref[...] = pre
    g = lng_ref[...]
    b = lnb_ref[...]
    ain_ref[...] = _ln_mm(pre, g, b, o_a)
    post = _dg(hidden, wpost_ref[...])
    aout_ref[...] = _ln_mm(post, g, b, o_a)

    gl = _dg(x, wrg_ref[...])
    ll = _dg(x, wre_ref[...])
    gp = jax.nn.softmax(gl, axis=-1)
    lp = jax.nn.softmax(ll, axis=-1)

    iog = lax.broadcasted_iota(jnp.int32, gp.shape, 1)
    cw = jnp.max(gp, axis=-1, keepdims=True)
    cg = jnp.min(jnp.where(gp == cw, iog, ng), axis=-1, keepdims=True)

    iol = lax.broadcasted_iota(jnp.int32, lp.shape, 1)
    v1 = jnp.max(lp, axis=-1, keepdims=True)
    i1 = jnp.min(jnp.where(lp == v1, iol, gs), axis=-1, keepdims=True)
    lp2 = jnp.where(iol == i1, -1.0, lp)
    v2 = jnp.max(lp2, axis=-1, keepdims=True)
    i2 = jnp.min(jnp.where(lp2 == v2, iol, gs), axis=-1, keepdims=True)

    lsum = v1 + v2 + 1e-7
    f1 = cw * v1 / lsum
    f2 = cw * v2 / lsum

    cols = lax.broadcasted_iota(jnp.int32, (x.shape[0], n_exp), 1)
    g_of = cols // gs
    j_of = cols % gs
    ew = jnp.where(
        g_of == cg,
        jnp.where(j_of == i1, f1, jnp.where(j_of == i2, f2, 0.0)),
        0.0)
    ew_ref[...] = ew

    @pl.when(i == 0)
    def _():
        load_acc[...] = jnp.zeros_like(load_acc)
        sq_acc[...] = jnp.zeros_like(sq_acc)

    load_acc[...] += jnp.sum(ew, axis=0, keepdims=True)
    zpart = (jnp.sum(gl * gl) / (n_tok * ng)
             + jnp.sum(ll * ll) / (n_tok * gs))
    sq_acc[...] += zpart.reshape(1, 1)

    @pl.when(i == nprog - 1)
    def _():
        load = load_acc[...]
        target = jnp.sum(load) / n_exp
        lb = jnp.sum((load - target) ** 2) / n_exp
        loss_ref[...] = 0.001 * (lb + sq_acc[...])


def _pass_b_body(hid_ref, pre_ref, aint_ref, ainf_ref, aoutf_ref, ew_ref,
                 wadp_ref, m16_ref, b16_ref, gflat_ref, bflat_ref,
                 wda_ref, woe_ref, wdown_ref, out_ref, *, n_exp, pad):
    h = hid_ref[0]
    ain_i = aint_ref[0]
    ain_b = ainf_ref[0]
    aout_b = aoutf_ref[0]
    ew = ew_ref[0]

    aw = _dg(ain_i, aout_b)
    aw = jax.nn.silu(jnp.clip(aw, -5.0, 5.0))
    ad = lax.dot_general(aw, ain_b, (((1,), (0,)), ((), ())),
                         preferred_element_type=jnp.float32)
    sh = _dg(h, wdown_ref[...]) + 0.1 * _dg(ad, wda_ref[...])

    pre = pre_ref[0]
    zp = _dg(pre, wadp_ref[...])
    m16 = m16_ref[...]
    b16 = b16_ref[...]
    m = _dg(zp, m16)
    e2 = _dg(zp * zp, m16)
    r = lax.rsqrt(e2 - m * m + 1e-5)
    mb = _dg(m, b16)
    rb = _dg(r, b16)
    ewb = _dg(ew, b16)
    wf = ((zp - mb) * rb * gflat_ref[...] + bflat_ref[...]) * ewb
    wacc = wf[:, 0:pad]
    for e in range(1, n_exp):
        wacc = wacc + wf[:, e * pad:(e + 1) * pad]
    contrib = _dg(wacc, woe_ref[...])

    ones_e = jnp.full((1, n_exp), 1.0, jnp.float32)
    wsum = _dg(ew, ones_e)
    out_ref[0] = sh * wsum + 0.1 * contrib


def kernel(x, W_up, W_gate, W_down, W_pre, W_post, ln_g, ln_b, W_ap, W_adp,
           lne_g, lne_b, W_ep, W_op, W_rg, W_re):
    B, S, D = x.shape
    H = W_up.shape[0]
    A = W_pre.shape[0]
    E = W_adp.shape[0]
    NG = W_rg.shape[0]
    GS = W_re.shape[0]
    N = B * S
    TA = 512
    TB = 512
    PAD = 128

    xf = x.reshape(N, D)
    lng2 = ln_g.reshape(1, A)
    lnb2 = ln_b.reshape(1, A)

    wadp_pad = jnp.pad(W_adp, ((0, 0), (0, PAD - A), (0, 0))).reshape(E * PAD, A)
    blk = jnp.arange(E * PAD) // PAD
    lane = jnp.arange(E * PAD) % PAD
    real = (lane < A).astype(jnp.float32)
    m16 = (jnp.arange(E)[:, None] == blk[None, :]).astype(jnp.float32)
    m16 = m16 * real[None, :] / A
    b16 = (blk[:, None] == jnp.arange(E)[None, :]).astype(jnp.float32)
    gflat = jnp.pad(lne_g, ((0, 0), (0, PAD - A))).reshape(1, E * PAD)
    bflat = jnp.pad(lne_b, ((0, 0), (0, PAD - A))).reshape(1, E * PAD)
    woe_padder = lambda w: jnp.pad(w, ((0, 0), (0, PAD - A)))

    const = lambda *_: (0, 0)
    wda, woe = pl.pallas_call(
        _fold_body,
        in_specs=[
            pl.BlockSpec((D, H), const),
            pl.BlockSpec((H, A), const),
            pl.BlockSpec((D, H), const),
            pl.BlockSpec((H, A), const),
        ],
        out_specs=[
            pl.BlockSpec((D, A), const),
            pl.BlockSpec((D, A), const),
        ],
        out_shape=[
            jax.ShapeDtypeStruct((D, A), jnp.float32),
            jax.ShapeDtypeStruct((D, A), jnp.float32),
        ],
    )(W_down, W_ap, W_op, W_ep)
    woe_pad = woe_padder(woe)

    hid, pre, ain, aout, ew, loss = pl.pallas_call(
        functools.partial(_pass_a_body, n_tok=N, ng=NG, gs=GS, n_exp=E),
        grid=(N // TA,),
        in_specs=[
            pl.BlockSpec((TA, D), lambda i: (i, 0)),
            pl.BlockSpec((H, D), const),
            pl.BlockSpec((H, D), const),
            pl.BlockSpec((A, D), const),
            pl.BlockSpec((A, H), const),
            pl.BlockSpec((1, A), const),
            pl.BlockSpec((1, A), const),
            pl.BlockSpec((NG, D), const),
            pl.BlockSpec((GS, D), const),
        ],
        out_specs=[
            pl.BlockSpec((TA, H), lambda i: (i, 0)),
            pl.BlockSpec((TA, A), lambda i: (i, 0)),
            pl.BlockSpec((TA, A), lambda i: (i, 0)),
            pl.BlockSpec((TA, A), lambda i: (i, 0)),
            pl.BlockSpec((TA, E), lambda i: (i, 0)),
            pl.BlockSpec((1, 1), const),
        ],
        out_shape=[
            jax.ShapeDtypeStruct((N, H), jnp.float32),
            jax.ShapeDtypeStruct((N, A), jnp.float32),
            jax.ShapeDtypeStruct((N, A), jnp.float32),
            jax.ShapeDtypeStruct((N, A), jnp.float32),
            jax.ShapeDtypeStruct((N, E), jnp.float32),
            jax.ShapeDtypeStruct((1, 1), jnp.float32),
        ],
        scratch_shapes=[
            pltpu.VMEM((1, E), jnp.float32),
            pltpu.VMEM((1, 1), jnp.float32),
        ],
    )(xf, W_up, W_gate, W_pre, W_post, lng2, lnb2, W_rg, W_re)

    hid3 = hid.reshape(B, S, H)
    pre3 = pre.reshape(B, S, A)
    ain3 = ain.reshape(B, S, A)
    aout3 = aout.reshape(B, S, A)
    ew3 = ew.reshape(B, S, E)

    const3 = lambda b, i: (0, 0)
    out3 = pl.pallas_call(
        functools.partial(_pass_b_body, n_exp=E, pad=PAD),
        grid=(B, S // TB),
        in_specs=[
            pl.BlockSpec((1, TB, H), lambda b, i: (b, i, 0)),
            pl.BlockSpec((1, TB, A), lambda b, i: (b, i, 0)),
            pl.BlockSpec((1, TB, A), lambda b, i: (b, i, 0)),
            pl.BlockSpec((1, S, A), lambda b, i: (b, 0, 0)),
            pl.BlockSpec((1, S, A), lambda b, i: (b, 0, 0)),
            pl.BlockSpec((1, TB, E), lambda b, i: (b, i, 0)),
            pl.BlockSpec((E * PAD, A), const3),
            pl.BlockSpec((E, E * PAD), const3),
            pl.BlockSpec((E * PAD, E), const3),
            pl.BlockSpec((1, E * PAD), const3),
            pl.BlockSpec((1, E * PAD), const3),
            pl.BlockSpec((D, A), const3),
            pl.BlockSpec((D, PAD), const3),
            pl.BlockSpec((D, H), const3),
        ],
        out_specs=pl.BlockSpec((1, TB, D), lambda b, i: (b, i, 0)),
        out_shape=jax.ShapeDtypeStruct((B, S, D), jnp.float32),
    )(hid3, pre3, ain3, ain3, aout3, ew3, wadp_pad, m16, b16, gflat, bflat,
      wda, woe_pad, W_down)

    return out3, loss[0, 0]
